# Initial kernel scaffold; baseline (speedup 1.0000x reference)
import jax
import jax.numpy as jnp
from jax.experimental import pallas as pl


def kernel(pos_N, pos_Ca, pos_C, residue_type, batch, edge_index, params):
    raise NotImplementedError("write your pallas kernel here")


# speedup vs baseline: 1.4263x; 1.4263x over previous
import functools

import jax
import jax.numpy as jnp
from jax.experimental import pallas as pl
from jax.experimental.pallas import tpu as pltpu

EMB = 128
NRAD = 64
CUTOFF = 5.0
GAMMA = 10.0
CH = 80


def _dense_body(S_ref, pN_ref, pCa_ref, pC_ref, rt_emb_ref,
                ligW_ref, ligb_ref, emb_ref, bbW_ref, bbb_ref,
                e01W_ref, e01b_ref, sW1_ref, sb1_ref, sW2_ref, sb2_ref,
                eW1_ref, eb1_ref, bng_ref, bnb_ref, eW2_ref, eb2_ref,
                out_ref):
    f32 = jnp.float32
    S = S_ref[...]
    ligW = ligW_ref[...]
    ligb = ligb_ref[...]
    emb = emb_ref[...]
    R = S.shape[1]

    xs = []
    for k in range(3):
        acc = jnp.zeros((R, EMB), f32)
        deg = jnp.zeros((R, 1), f32)
        for t in range(3):
            Sg = S[k * 3 + t]
            rbf_sum = Sg[:, :NRAD]
            cnt = Sg[:, NRAD:NRAD + 1]
            sw = jnp.dot(rbf_sum, ligW, preferred_element_type=f32) + cnt * ligb
            acc = acc + sw * emb[t:t + 1, :]
            deg = deg + cnt
        acc = acc + deg * emb[k:k + 1, :]
        xs.append(acc)

    pN = pN_ref[...]
    pCa = pCa_ref[...]
    pC = pC_ref[...]

    v1 = pCa - pN
    d1sq = jnp.sum(v1 * v1, axis=1, keepdims=True)
    d1 = jnp.sqrt(d1sq + 1e-12)
    f1 = v1 / jnp.maximum(d1, 1e-8)
    v2 = pC - pCa
    d2sq = jnp.sum(v2 * v2, axis=1, keepdims=True)
    d2 = jnp.sqrt(d2sq + 1e-12)
    f2 = v2 / jnp.maximum(d2, 1e-8)
    cx = f1[:, 1:2] * f2[:, 2:3] - f1[:, 2:3] * f2[:, 1:2]
    cy = f1[:, 2:3] * f2[:, 0:1] - f1[:, 0:1] * f2[:, 2:3]
    cz = f1[:, 0:1] * f2[:, 1:2] - f1[:, 1:2] * f2[:, 0:1]
    cn = jnp.sqrt(cx * cx + cy * cy + cz * cz + 1e-12)
    cn = jnp.maximum(cn, 1e-8)
    f3 = jnp.concatenate([cx / cn, cy / cn, cz / cn], axis=1)

    bbW = bbW_ref[...]
    bbb = bbb_ref[...]
    e01W = e01W_ref[...]
    e01b = e01b_ref[...]
    eW1 = eW1_ref[...]
    eb1 = eb1_ref[...]
    bng = bng_ref[...]
    bnb = bnb_ref[...]
    eW2 = eW2_ref[...]
    eb2 = eb2_ref[...]

    centers = jax.lax.broadcasted_iota(jnp.int32, (R, NRAD), 1).astype(f32) * (
        CUTOFF / (NRAD - 1))

    def backbone_edge(dist_col, vec, repr_u, repr_v):
        rbf = jnp.exp(-GAMMA * (dist_col - centers) ** 2)
        radial = jnp.dot(rbf, bbW, preferred_element_type=f32) + bbb
        base = radial * repr_u * repr_v
        Ed = [jnp.dot(vec[:, d:d + 1] * base, e01W,
                      preferred_element_type=f32) + e01b for d in range(3)]
        frames = (f1, f2, f3)
        scal = []
        for dp in range(3):
            fr = frames[dp]
            s = (Ed[0] * fr[:, 0:1] + Ed[1] * fr[:, 1:2]
                 + Ed[2] * fr[:, 2:3])
            scal.append(s)
        y = jnp.zeros((R, EMB), f32)
        for m in range(32):
            a = (scal[0] * sW1_ref[0, m] + scal[1] * sW1_ref[1, m]
                 + scal[2] * sW1_ref[2, m] + sb1_ref[m])
            y = y + sW2_ref[m, 0] * (a * jax.nn.sigmoid(a))
        y = y + sb2_ref[0] + scal[0]
        h2 = jnp.dot(y, eW1, preferred_element_type=f32) + eb1
        h2 = h2 * bng + bnb
        h2 = jnp.maximum(h2, 0.0)
        return jnp.dot(h2, eW2, preferred_element_type=f32) + eb2

    s1 = backbone_edge(d1, v1, xs[0], xs[1])
    s2 = backbone_edge(d2, v2, xs[1], xs[2])
    out_ref[...] = (s1 + s2) * 0.5 + rt_emb_ref[...]


def _dense_forward(S, pN, pCa, pC, rt_emb, p, interpret=False):
    n = pN.shape[0]
    R = 1000
    grid = (n // R,)
    f32 = jnp.float32

    def rb(i):
        return (i, 0)

    def full2(i):
        return (0, 0)

    def full3(i):
        return (0, 0, 0)

    in_specs = [
        pl.BlockSpec((9, R, CH), lambda i: (0, i, 0)),
        pl.BlockSpec((R, 3), rb),
        pl.BlockSpec((R, 3), rb),
        pl.BlockSpec((R, 3), rb),
        pl.BlockSpec((R, EMB), rb),
        pl.BlockSpec((NRAD, EMB), full2),
        pl.BlockSpec((1, EMB), full2),
        pl.BlockSpec((3, EMB), full2),
        pl.BlockSpec((NRAD, EMB), full2),
        pl.BlockSpec((1, EMB), full2),
        pl.BlockSpec((EMB, EMB), full2),
        pl.BlockSpec((1, EMB), full2),
        pl.BlockSpec(memory_space=pltpu.SMEM),
        pl.BlockSpec(memory_space=pltpu.SMEM),
        pl.BlockSpec(memory_space=pltpu.SMEM),
        pl.BlockSpec(memory_space=pltpu.SMEM),
        pl.BlockSpec((EMB, 2 * EMB), full2),
        pl.BlockSpec((1, 2 * EMB), full2),
        pl.BlockSpec((1, 2 * EMB), full2),
        pl.BlockSpec((1, 2 * EMB), full2),
        pl.BlockSpec((2 * EMB, EMB), full2),
        pl.BlockSpec((1, EMB), full2),
    ]
    return pl.pallas_call(
        _dense_body,
        grid=grid,
        in_specs=in_specs,
        out_specs=pl.BlockSpec((R, EMB), rb),
        out_shape=jax.ShapeDtypeStruct((n, EMB), f32),
        interpret=interpret,
    )(S, pN, pCa, pC, rt_emb,
      p['lig_rbf_W'], p['lig_rbf_b'].reshape(1, EMB), p['backbone_emb'],
      p['bb_rbf_W'], p['bb_rbf_b'].reshape(1, EMB),
      p['edge01_W'], p['edge01_b'].reshape(1, EMB),
      p['scal_W1'], p['scal_b1'], p['scal_W2'], p['scal_b2'],
      p['edge_W1'], p['edge_b1'].reshape(1, 2 * EMB),
      p['edge_bn_gamma'].reshape(1, 2 * EMB), p['edge_bn_beta'].reshape(1, 2 * EMB),
      p['edge_W2'], p['edge_b2'].reshape(1, EMB))


def _edge_accum_xla(pos, edge_index, n):
    i, j = edge_index[0], edge_index[1]
    vec = pos[i] - pos[j]
    dist = jnp.sqrt(jnp.sum(vec * vec, -1) + 1e-12)
    centers = jnp.linspace(0.0, CUTOFF, NRAD)
    rbf = jnp.exp(-GAMMA * (dist[:, None] - centers) ** 2)
    rbfp = jnp.concatenate([rbf, jnp.ones((rbf.shape[0], 1), rbf.dtype)], -1)
    g = ((i % 3) * 3 + (j % 3)) * n + i // 3
    S = jax.ops.segment_sum(rbfp, g, num_segments=9 * n)
    S = jnp.pad(S, ((0, 0), (0, CH - NRAD - 1)))
    return S.reshape(9, n, CH)


def kernel(pos_N, pos_Ca, pos_C, residue_type, batch, edge_index, params):
    p = params
    n = residue_type.shape[0]
    pos = jnp.stack([pos_N, pos_Ca, pos_C], axis=1).reshape(-1, 3)
    S = _edge_accum_xla(pos, edge_index, n)
    rt_emb = p['residue_emb'][residue_type]
    return _dense_forward(S, pos_N, pos_Ca, pos_C, rt_emb, p)
